# Initial kernel scaffold; baseline (speedup 1.0000x reference)
import jax
import jax.numpy as jnp
from jax.experimental import pallas as pl


def kernel(x1, edge_index1, x2, edge_index2, W1, b1, W2, b2):
    raise NotImplementedError("write your pallas kernel here")


# speedup vs baseline: 2.5958x; 2.5958x over previous
import functools

import jax
import jax.numpy as jnp
from jax import lax
from jax.experimental import pallas as pl
from jax.experimental.pallas import tpu as pltpu
from jax.experimental.pallas import tpu_sc as plsc

N = 10000
NP = 10240
NS = 2 * NP
E = 160000
EP = 163840
ES = 2 * EP
NTILES = 32
PER_TILE = ES // NTILES
EB = 128
NBB = PER_TILE // EB
FC = 128
NCH = 4
RPT = NP // 16
IN_DIM = 256
HID = 512


DR = NP // 128


def _deg_body(dst1_hbm, zer_hbm, out_hbm, dstraw, hist, idv, acc, sem):
    cid = lax.axis_index("c")
    sid = lax.axis_index("s")
    wid = cid * 16 + sid
    pltpu.sync_copy(dst1_hbm.at[pl.ds(wid * PER_TILE, PER_TILE)], dstraw)
    pltpu.sync_copy(zer_hbm, hist)

    @pl.when(sid < DR // 8)
    def _():
        pltpu.sync_copy(zer_hbm.at[pl.ds(0, 8)], acc.at[pl.ds(sid * 8, 8)])
    for j in range(DR // 16):
        idv[pl.ds(j * 16, 16)] = (
            lax.broadcasted_iota(jnp.int32, (16,), 0) + j * 16)
    one = jnp.full((16,), 1.0, jnp.float32)

    def body(e, carry):
        d = dstraw[pl.ds(e * 16, 16)]
        plsc.addupdate_scatter(hist, [d >> 7, d & 127], one)
        return carry

    lax.fori_loop(0, PER_TILE // 16, body, 0)
    plsc.subcore_barrier()
    pltpu.sync_copy(hist, acc.at[idv], add=True)
    plsc.subcore_barrier()

    @pl.when(sid < DR // 8)
    def _():
        pltpu.sync_copy(acc.at[pl.ds(sid * 8, 8)],
                        out_hbm.at[cid, pl.ds(sid * 8, 8)])


@functools.cache
def _deg_call_fn():
    return pl.kernel(
        _deg_body,
        out_type=jax.ShapeDtypeStruct((2, DR, 128), jnp.float32),
        mesh=plsc.VectorSubcoreMesh(core_axis_name="c", subcore_axis_name="s"),
        compiler_params=pltpu.CompilerParams(needs_layout_passes=False),
        scratch_types=[
            pltpu.VMEM((PER_TILE,), jnp.int32),
            pltpu.VMEM((DR, 128), jnp.float32),
            pltpu.VMEM((DR,), jnp.int32),
            pltpu.VMEM_SHARED((DR, 128), jnp.float32),
            pltpu.SemaphoreType.DMA,
        ],
    )


NH = NP // 2
ACCR = NH + 8
RPH = NH // 16


def _seg_body(src_hbm, dst1_hbm, tab_hbm, zer_hbm, out_hbm,
              srcv, dstraw, idxh, buf0, buf1, acc, sem0, sem1):
    cid = lax.axis_index("c")
    sid = lax.axis_index("s")
    wid = cid * 16 + sid
    pltpu.sync_copy(src_hbm.at[pl.ds(wid * NBB, NBB)], srcv)
    pltpu.sync_copy(dst1_hbm.at[pl.ds(wid * PER_TILE, PER_TILE)], dstraw)

    def remap(b, carry):
        for h in range(2):
            for j in range(EB // 16):
                d = dstraw[pl.ds(b * EB + j * 16, 16)]
                t = d - h * NH
                ok = jnp.logical_and(t >= 0, t < NH)
                idxh[h, b, pl.ds(j * 16, 16)] = jnp.where(ok, t, NH)
        return carry

    lax.fori_loop(0, NBB, remap, 0)

    for c in range(NCH):
        tab_c = tab_hbm.at[c]
        for h in range(2):
            idx_h = idxh.at[h]
            pltpu.sync_copy(zer_hbm, acc.at[pl.ds(sid * RPH, RPH)])
            plsc.subcore_barrier()
            pltpu.async_copy(tab_c.at[srcv.at[0]], buf0, sem0)

            def body(i, carry):
                b = 2 * i
                pltpu.async_copy(tab_c.at[srcv.at[b + 1]], buf1, sem1)
                pltpu.make_async_copy(tab_c.at[srcv.at[b]], buf0, sem0).wait()
                pltpu.sync_copy(buf0, acc.at[idx_h.at[b]], add=True)

                @pl.when(b + 2 < NBB)
                def _():
                    pltpu.async_copy(tab_c.at[srcv.at[b + 2]], buf0, sem0)

                pltpu.make_async_copy(tab_c.at[srcv.at[b + 1]], buf1,
                                      sem1).wait()
                pltpu.sync_copy(buf1, acc.at[idx_h.at[b + 1]], add=True)
                return carry

            lax.fori_loop(0, NBB // 2, body, 0)
            plsc.subcore_barrier()
            pltpu.sync_copy(
                acc.at[pl.ds(sid * RPH, RPH)],
                out_hbm.at[c, pl.ds(cid * NP + h * NH + sid * RPH, RPH)])
            plsc.subcore_barrier()


@functools.cache
def _seg_call_fn():
    return pl.kernel(
        _seg_body,
        out_type=jax.ShapeDtypeStruct((NCH, NS, FC), jnp.float32),
        mesh=plsc.VectorSubcoreMesh(core_axis_name="c", subcore_axis_name="s"),
        scratch_types=[
            pltpu.VMEM((NBB, EB), jnp.int32),
            pltpu.VMEM((PER_TILE,), jnp.int32),
            pltpu.VMEM((2, NBB, EB), jnp.int32),
            pltpu.VMEM((EB, FC), jnp.float32),
            pltpu.VMEM((EB, FC), jnp.float32),
            pltpu.VMEM_SHARED((ACCR, FC), jnp.float32),
            pltpu.SemaphoreType.DMA,
            pltpu.SemaphoreType.DMA,
        ],
    )


_MB = 256


def _mm_body(x_ref, w_ref, dinv_ref, out_ref):
    y = jnp.dot(x_ref[...], w_ref[...], preferred_element_type=jnp.float32)
    y = y * dinv_ref[...]
    for c in range(NCH):
        out_ref[c] = y[:, c * FC:(c + 1) * FC]


def _mm_scale(x, w, dinv):
    k = x.shape[1]
    return pl.pallas_call(
        _mm_body,
        grid=(NS // _MB,),
        in_specs=[
            pl.BlockSpec((_MB, k), lambda m: (m, 0)),
            pl.BlockSpec((k, HID), lambda m: (0, 0)),
            pl.BlockSpec((_MB, 1), lambda m: (m, 0)),
        ],
        out_specs=pl.BlockSpec((NCH, _MB, FC), lambda m: (0, m, 0)),
        out_shape=jax.ShapeDtypeStruct((NCH, NS, FC), jnp.float32),
        compiler_params=pltpu.CompilerParams(
            dimension_semantics=("arbitrary",)),
    )(x, w, dinv)


def _combine_body(relu, s_ref, hw_ref, dinv_ref, b_ref, out_ref):
    m = pl.program_id(0)
    rows = m * _MB + lax.broadcasted_iota(jnp.int32, (_MB, FC), 0)
    valid = (rows % NP) < N
    dinv = dinv_ref[...]
    for c in range(NCH):
        col = s_ref[c] + hw_ref[c]
        col = dinv * col + b_ref[0, c * FC:(c + 1) * FC]
        if relu:
            col = jnp.maximum(col, 0.0)
        out_ref[:, c * FC:(c + 1) * FC] = jnp.where(valid, col, 0.0)


def _combine(s, hw, dinv, bias, relu):
    return pl.pallas_call(
        functools.partial(_combine_body, relu),
        grid=(NS // _MB,),
        in_specs=[
            pl.BlockSpec((NCH, _MB, FC), lambda m: (0, m, 0)),
            pl.BlockSpec((NCH, _MB, FC), lambda m: (0, m, 0)),
            pl.BlockSpec((_MB, 1), lambda m: (m, 0)),
            pl.BlockSpec((1, HID), lambda m: (0, 0)),
        ],
        out_specs=pl.BlockSpec((_MB, HID), lambda m: (m, 0)),
        out_shape=jax.ShapeDtypeStruct((NS, HID), jnp.float32),
        compiler_params=pltpu.CompilerParams(
            dimension_semantics=("arbitrary",)),
    )(s, hw, dinv, bias)


def _stats_body(o_ref, out_ref):
    mm = pl.program_id(1)
    blk = o_ref[...]

    @pl.when(mm == 0)
    def _():
        out_ref[...] = jnp.zeros_like(out_ref)

    out_ref[0, 0, :] += jnp.sum(blk, axis=0)
    out_ref[0, 1, :] += jnp.sum(blk * blk, axis=0)


def _stats(o):
    return pl.pallas_call(
        _stats_body,
        grid=(2, NP // _MB),
        in_specs=[pl.BlockSpec((_MB, HID), lambda g, mm: (g * (NP // _MB) + mm, 0))],
        out_specs=pl.BlockSpec((1, 2, HID), lambda g, mm: (g, 0, 0)),
        out_shape=jax.ShapeDtypeStruct((2, 2, HID), jnp.float32),
        compiler_params=pltpu.CompilerParams(
            dimension_semantics=("arbitrary", "arbitrary")),
    )(o)


def _norm_body(o_ref, st_ref, out_ref):
    s = st_ref[0, 0, :]
    ss = st_ref[0, 1, :]
    mean = s / N
    var = (ss - N * mean * mean) / (N - 1)
    out_ref[...] = (o_ref[...] - mean) / jnp.sqrt(var)


def _norm(o, st):
    return pl.pallas_call(
        _norm_body,
        grid=(NS // _MB,),
        in_specs=[
            pl.BlockSpec((_MB, HID), lambda m: (m, 0)),
            pl.BlockSpec((1, 2, HID), lambda m: (m // (NP // _MB), 0, 0)),
        ],
        out_specs=pl.BlockSpec((_MB, HID), lambda m: (m, 0)),
        out_shape=jax.ShapeDtypeStruct((NS, HID), jnp.float32),
        compiler_params=pltpu.CompilerParams(
            dimension_semantics=("arbitrary",)),
    )(o, st)


def _pad_edges(a, fill):
    return jnp.concatenate([a, jnp.full((EP - E,), fill, jnp.int32)])


def kernel(x1, edge_index1, x2, edge_index2, W1, b1, W2, b2):
    x = jnp.zeros((NS, IN_DIM), jnp.float32)
    x = x.at[0:N].set(x1).at[NP:NP + N].set(x2)
    src = jnp.concatenate([_pad_edges(edge_index1[0], NP - 1),
                           _pad_edges(edge_index2[0] + NP, NS - 1)])
    dst = jnp.concatenate([_pad_edges(edge_index1[1], NP - 1),
                           _pad_edges(edge_index2[1], NP - 1)])
    src3 = src.reshape(ES // EB, EB)
    zer80 = jnp.zeros((DR, 128), jnp.float32)
    zerfc = jnp.zeros((RPH, FC), jnp.float32)
    b1r = b1.reshape(1, HID)
    b2r = b2.reshape(1, HID)

    degp = _deg_call_fn()(dst, zer80)
    dinv = lax.rsqrt(1.0 + degp.reshape(NS)).reshape(NS, 1)

    hw1 = _mm_scale(x, W1, dinv)
    s1 = _seg_call_fn()(src3, dst, hw1, zerfc)
    h = _combine(s1, hw1, dinv, b1r, True)

    hw2 = _mm_scale(h, W2, dinv)
    s2 = _seg_call_fn()(src3, dst, hw2, zerfc)
    o = _combine(s2, hw2, dinv, b2r, False)

    st = _stats(o)
    z = _norm(o, st)
    return z[0:N], z[NP:NP + N]


# speedup vs baseline: 4.3275x; 1.6672x over previous
import functools

import jax
import jax.numpy as jnp
from jax import lax
from jax.experimental import pallas as pl
from jax.experimental.pallas import tpu as pltpu
from jax.experimental.pallas import tpu_sc as plsc

N = 10000
NP = 10240
NS = 2 * NP
E = 160000
EP = 163840
ES = 2 * EP
NTILES = 32
PER_TILE = ES // NTILES
EB = 128
NBB = PER_TILE // EB
FC = 128
NCH = 4
IN_DIM = 256
HID = 512


DR = NP // 128

NH = 2048
NPASS = NP // NH
ACCR = NH + 8
RPH = NH // 16


def _deg_body(dst2_hbm, zer_hbm, out_hbm, dstraw, hist, idv, acc, sem):
    cid = lax.axis_index("c")
    sid = lax.axis_index("s")
    wid = cid * 16 + sid
    pltpu.sync_copy(dst2_hbm.at[0].at[pl.ds(wid * NBB, NBB)], dstraw)
    pltpu.sync_copy(zer_hbm, hist)

    @pl.when(sid < DR // 8)
    def _():
        pltpu.sync_copy(zer_hbm.at[pl.ds(0, 8)], acc.at[pl.ds(sid * 8, 8)])

    for j in range(DR // 16):
        idv[pl.ds(j * 16, 16)] = (
            lax.broadcasted_iota(jnp.int32, (16,), 0) + j * 16)
    one = jnp.full((16,), 1.0, jnp.float32)

    def body(e, carry):
        for j in range(EB // 16):
            d = dstraw[e, pl.ds(j * 16, 16)]
            plsc.addupdate_scatter(hist, [d >> 7, d & 127], one)
        return carry

    lax.fori_loop(0, NBB, body, 0)
    plsc.subcore_barrier()
    pltpu.sync_copy(hist, acc.at[idv], add=True)
    plsc.subcore_barrier()

    @pl.when(sid < DR // 8)
    def _():
        pltpu.sync_copy(acc.at[pl.ds(sid * 8, 8)],
                        out_hbm.at[cid, pl.ds(sid * 8, 8)])


@functools.cache
def _deg_call_fn():
    return pl.kernel(
        _deg_body,
        out_type=jax.ShapeDtypeStruct((2, DR, 128), jnp.float32),
        mesh=plsc.VectorSubcoreMesh(core_axis_name="c", subcore_axis_name="s"),
        compiler_params=pltpu.CompilerParams(needs_layout_passes=False),
        scratch_types=[
            pltpu.VMEM((NBB, EB), jnp.int32),
            pltpu.VMEM((DR, 128), jnp.float32),
            pltpu.VMEM((DR,), jnp.int32),
            pltpu.VMEM_SHARED((DR, 128), jnp.float32),
            pltpu.SemaphoreType.DMA,
        ],
    )


def _seg_body(src2_hbm, dst2_hbm, tab_hbm, zer_hbm, out_hbm,
              srcraw, dstraw, srcc, dstc, srcc2, idx2, buf0, buf1, acc,
              sem0, sem1):
    cid = lax.axis_index("c")
    sid = lax.axis_index("s")
    wid = cid * 16 + sid
    pltpu.sync_copy(src2_hbm.at[0].at[pl.ds(wid * NBB, NBB)], srcraw)
    pltpu.sync_copy(dst2_hbm.at[0].at[pl.ds(wid * NBB, NBB)], dstraw)

    def count(e, carry):
        for j in range(EB // 16):
            d = dstraw[e, pl.ds(j * 16, 16)]
            q = d >> 11
            carry = tuple(
                carry[k] + jnp.sum((q == k).astype(jnp.int32))
                for k in range(NPASS))
        return carry

    cnts = lax.fori_loop(0, NBB, count, (jnp.int32(0),) * NPASS)
    offs = [jnp.int32(0)]
    for k in range(NPASS):
        offs.append(offs[k] + cnts[k])

    def compact(e, carry):
        for j in range(EB // 16):
            s = srcraw[e, pl.ds(j * 16, 16)]
            d = dstraw[e, pl.ds(j * 16, 16)]
            q = d >> 11
            nc = []
            for k in range(NPASS):
                m = q == k
                plsc.store_compressed(srcc.at[pl.ds(carry[k], 16)], s,
                                      mask=m)
                plsc.store_compressed(dstc.at[pl.ds(carry[k], 16)], d,
                                      mask=m)
                nc.append(carry[k] + jnp.sum(m.astype(jnp.int32)))
            carry = tuple(nc)
        return carry

    lax.fori_loop(0, NBB, compact, tuple(offs[:NPASS]))

    def s2d(b, carry):
        for j in range(EB // 16):
            srcc2[b, pl.ds(j * 16, 16)] = srcc[pl.ds(b * EB + j * 16, 16)]
        return carry

    lax.fori_loop(0, NBB, s2d, 0)

    for q in range(NPASS):
        lob = offs[q] // EB
        hib = (offs[q + 1] + EB - 1) // EB

        def remap(b, carry):
            for j in range(EB // 16):
                d = dstc[pl.ds(b * EB + j * 16, 16)]
                t = d - q * NH
                ok = jnp.logical_and(t >= 0, t < NH)
                idx2[b, pl.ds(j * 16, 16)] = jnp.where(ok, t, NH)
            return carry

        lax.fori_loop(lob, hib, remap, 0)

        def act(b):
            return jnp.logical_and(b >= lob, b < hib)

        for c in range(NCH):
            tab_c = tab_hbm.at[c]
            pltpu.sync_copy(zer_hbm, acc.at[pl.ds(sid * RPH, RPH)])
            plsc.subcore_barrier()

            @pl.when(act(0))
            def _():
                pltpu.async_copy(tab_c.at[srcc2.at[0]], buf0, sem0)

            def body(i, carry):
                b = 2 * i

                @pl.when(act(b + 1))
                def _():
                    pltpu.async_copy(tab_c.at[srcc2.at[b + 1]], buf1, sem1)

                @pl.when(act(b))
                def _():
                    pltpu.make_async_copy(tab_c.at[srcc2.at[b]], buf0,
                                          sem0).wait()
                    pltpu.sync_copy(buf0, acc.at[idx2.at[b]], add=True)

                @pl.when(act(b + 2))
                def _():
                    pltpu.async_copy(tab_c.at[srcc2.at[b + 2]], buf0, sem0)

                @pl.when(act(b + 1))
                def _():
                    pltpu.make_async_copy(tab_c.at[srcc2.at[b + 1]], buf1,
                                          sem1).wait()
                    pltpu.sync_copy(buf1, acc.at[idx2.at[b + 1]], add=True)

                return carry

            lax.fori_loop(0, NBB // 2, body, 0)
            plsc.subcore_barrier()
            pltpu.sync_copy(
                acc.at[pl.ds(sid * RPH, RPH)],
                out_hbm.at[c, pl.ds(cid * NP + q * NH + sid * RPH, RPH)])
            plsc.subcore_barrier()


@functools.cache
def _seg_call_fn():
    return pl.kernel(
        _seg_body,
        out_type=jax.ShapeDtypeStruct((NCH, NS, FC), jnp.float32),
        mesh=plsc.VectorSubcoreMesh(core_axis_name="c", subcore_axis_name="s"),
        compiler_params=pltpu.CompilerParams(needs_layout_passes=False),
        scratch_types=[
            pltpu.VMEM((NBB, EB), jnp.int32),
            pltpu.VMEM((NBB, EB), jnp.int32),
            pltpu.VMEM((PER_TILE,), jnp.int32),
            pltpu.VMEM((PER_TILE,), jnp.int32),
            pltpu.VMEM((NBB, EB), jnp.int32),
            pltpu.VMEM((NBB, EB), jnp.int32),
            pltpu.VMEM((EB, FC), jnp.float32),
            pltpu.VMEM((EB, FC), jnp.float32),
            pltpu.VMEM_SHARED((ACCR, FC), jnp.float32),
            pltpu.SemaphoreType.DMA,
            pltpu.SemaphoreType.DMA,
        ],
    )


_MB = 256


def _mm_body(x_ref, w_ref, dinv_ref, out_ref):
    y = jnp.dot(x_ref[...], w_ref[...], preferred_element_type=jnp.float32)
    y = y * dinv_ref[...]
    for c in range(NCH):
        out_ref[c] = y[:, c * FC:(c + 1) * FC]


def _mm_scale(x, w, dinv):
    k = x.shape[1]
    return pl.pallas_call(
        _mm_body,
        grid=(NS // _MB,),
        in_specs=[
            pl.BlockSpec((_MB, k), lambda m: (m, 0)),
            pl.BlockSpec((k, HID), lambda m: (0, 0)),
            pl.BlockSpec((_MB, 1), lambda m: (m, 0)),
        ],
        out_specs=pl.BlockSpec((NCH, _MB, FC), lambda m: (0, m, 0)),
        out_shape=jax.ShapeDtypeStruct((NCH, NS, FC), jnp.float32),
        compiler_params=pltpu.CompilerParams(
            dimension_semantics=("arbitrary",)),
    )(x, w, dinv)


def _combine_body(relu, s_ref, hw_ref, dinv_ref, b_ref, out_ref):
    m = pl.program_id(0)
    rows = m * _MB + lax.broadcasted_iota(jnp.int32, (_MB, FC), 0)
    valid = (rows % NP) < N
    dinv = dinv_ref[...]
    for c in range(NCH):
        col = s_ref[c] + hw_ref[c]
        col = dinv * col + b_ref[0, c * FC:(c + 1) * FC]
        if relu:
            col = jnp.maximum(col, 0.0)
        out_ref[:, c * FC:(c + 1) * FC] = jnp.where(valid, col, 0.0)


def _combine(s, hw, dinv, bias, relu):
    return pl.pallas_call(
        functools.partial(_combine_body, relu),
        grid=(NS // _MB,),
        in_specs=[
            pl.BlockSpec((NCH, _MB, FC), lambda m: (0, m, 0)),
            pl.BlockSpec((NCH, _MB, FC), lambda m: (0, m, 0)),
            pl.BlockSpec((_MB, 1), lambda m: (m, 0)),
            pl.BlockSpec((1, HID), lambda m: (0, 0)),
        ],
        out_specs=pl.BlockSpec((_MB, HID), lambda m: (m, 0)),
        out_shape=jax.ShapeDtypeStruct((NS, HID), jnp.float32),
        compiler_params=pltpu.CompilerParams(
            dimension_semantics=("arbitrary",)),
    )(s, hw, dinv, bias)


def _stats_body(o_ref, out_ref):
    mm = pl.program_id(1)
    blk = o_ref[...]

    @pl.when(mm == 0)
    def _():
        out_ref[...] = jnp.zeros_like(out_ref)

    out_ref[0, 0, :] += jnp.sum(blk, axis=0)
    out_ref[0, 1, :] += jnp.sum(blk * blk, axis=0)


def _stats(o):
    return pl.pallas_call(
        _stats_body,
        grid=(2, NP // _MB),
        in_specs=[pl.BlockSpec((_MB, HID),
                               lambda g, mm: (g * (NP // _MB) + mm, 0))],
        out_specs=pl.BlockSpec((1, 2, HID), lambda g, mm: (g, 0, 0)),
        out_shape=jax.ShapeDtypeStruct((2, 2, HID), jnp.float32),
        compiler_params=pltpu.CompilerParams(
            dimension_semantics=("arbitrary", "arbitrary")),
    )(o)


def _norm_body(o_ref, st_ref, out_ref):
    s = st_ref[0, 0, :]
    ss = st_ref[0, 1, :]
    mean = s / N
    var = (ss - N * mean * mean) / (N - 1)
    out_ref[...] = (o_ref[...] - mean) / jnp.sqrt(var)


def _norm(o, st):
    return pl.pallas_call(
        _norm_body,
        grid=(NS // _MB,),
        in_specs=[
            pl.BlockSpec((_MB, HID), lambda m: (m, 0)),
            pl.BlockSpec((1, 2, HID), lambda m: (m // (NP // _MB), 0, 0)),
        ],
        out_specs=pl.BlockSpec((_MB, HID), lambda m: (m, 0)),
        out_shape=jax.ShapeDtypeStruct((NS, HID), jnp.float32),
        compiler_params=pltpu.CompilerParams(
            dimension_semantics=("arbitrary",)),
    )(o, st)


def _pad_edges(a, fill):
    return jnp.concatenate([a, jnp.full((EP - E,), fill, jnp.int32)])


def kernel(x1, edge_index1, x2, edge_index2, W1, b1, W2, b2):
    x = jnp.zeros((NS, IN_DIM), jnp.float32)
    x = x.at[0:N].set(x1).at[NP:NP + N].set(x2)
    src = jnp.concatenate([_pad_edges(edge_index1[0], NP - 1),
                           _pad_edges(edge_index2[0] + NP, NS - 1)])
    dst = jnp.concatenate([_pad_edges(edge_index1[1], NP - 1),
                           _pad_edges(edge_index2[1], NP - 1)])
    src2 = src.reshape(1, ES // EB, EB)
    dst2 = dst.reshape(1, ES // EB, EB)
    zer80 = jnp.zeros((DR, 128), jnp.float32)
    zerfc = jnp.zeros((RPH, FC), jnp.float32)
    b1r = b1.reshape(1, HID)
    b2r = b2.reshape(1, HID)

    degp = _deg_call_fn()(dst2, zer80)
    dinv = lax.rsqrt(1.0 + degp.reshape(NS)).reshape(NS, 1)

    hw1 = _mm_scale(x, W1, dinv)
    s1 = _seg_call_fn()(src2, dst2, hw1, zerfc)
    h = _combine(s1, hw1, dinv, b1r, True)

    hw2 = _mm_scale(h, W2, dinv)
    s2 = _seg_call_fn()(src2, dst2, hw2, zerfc)
    o = _combine(s2, hw2, dinv, b2r, False)

    st = _stats(o)
    z = _norm(o, st)
    return z[0:N], z[NP:NP + N]


# speedup vs baseline: 4.4402x; 1.0260x over previous
import functools

import jax
import jax.numpy as jnp
from jax import lax
from jax.experimental import pallas as pl
from jax.experimental.pallas import tpu as pltpu
from jax.experimental.pallas import tpu_sc as plsc

N = 10000
NP = 10240
NS = 2 * NP
E = 160000
EP = 163840
ES = 2 * EP
NTILES = 32
PER_TILE = ES // NTILES
EB = 128
NBB = PER_TILE // EB
LB = 128
NLB = PER_TILE // LB
WC = 128
NCH = 4
IN_DIM = 256
HID = 512


DR = NP // 128

NH = 2048
NHS = 11
NPASS = NP // NH
ACCR = NH + 8
RPH = NH // 16


def _deg_body(pk2_hbm, zer_hbm, out_hbm, pkraw, hist, sem):
    cid = lax.axis_index("c")
    sid = lax.axis_index("s")
    wid = cid * 16 + sid
    pltpu.sync_copy(pk2_hbm.at[0].at[pl.ds(wid * NLB, NLB)], pkraw)
    pltpu.sync_copy(zer_hbm.at[0], hist)
    one = jnp.full((16,), 1.0, jnp.float32)

    def body(e, carry):
        for j in range(LB // 16):
            d = pkraw[e, pl.ds(j * 16, 16)] >> 16
            plsc.addupdate_scatter(hist, [d >> 7, d & 127], one)
        return carry

    lax.fori_loop(0, NLB, body, 0)
    pltpu.sync_copy(hist, out_hbm.at[cid, sid])


@functools.cache
def _deg_call_fn():
    return pl.kernel(
        _deg_body,
        out_type=jax.ShapeDtypeStruct((2, 16, DR, 128), jnp.float32),
        mesh=plsc.VectorSubcoreMesh(core_axis_name="c", subcore_axis_name="s"),
        compiler_params=pltpu.CompilerParams(needs_layout_passes=False),
        scratch_types=[
            pltpu.VMEM((NLB, LB), jnp.int32),
            pltpu.VMEM((DR, 128), jnp.float32),
            pltpu.SemaphoreType.DMA,
        ],
    )


def _seg_body(pk2_hbm, tab_hbm, zer_hbm, out_hbm,
              pkraw, srcc, dstc, idx2, offsm, buf0, buf1, buf2, buf3, acc,
              sem0, sem1, sem2, sem3):
    cid = lax.axis_index("c")
    sid = lax.axis_index("s")
    wid = cid * 16 + sid
    pltpu.sync_copy(pk2_hbm.at[0].at[pl.ds(wid * NLB, NLB)], pkraw)

    def count(e, carry):
        for j in range(LB // 16):
            d = pkraw[e, pl.ds(j * 16, 16)] >> 16
            q = d >> NHS
            carry = tuple(
                carry[k] + jnp.sum((q == k).astype(jnp.int32))
                for k in range(NPASS))
        return carry

    cnts = lax.fori_loop(0, NLB, count, (jnp.int32(0),) * NPASS)
    offs = [jnp.int32(0)]
    for k in range(NPASS):
        offs.append(offs[k] + cnts[k])

    def compact(e, carry):
        for j in range(LB // 16):
            p = pkraw[e, pl.ds(j * 16, 16)]
            s = p & 0xFFFF
            d = p >> 16
            q = d >> NHS
            nc = []
            for k in range(NPASS):
                m = q == k
                plsc.store_compressed(srcc.at[pl.ds(carry[k], 16)], s,
                                      mask=m)
                plsc.store_compressed(dstc.at[pl.ds(carry[k], 16)], d,
                                      mask=m)
                nc.append(carry[k] + jnp.sum(m.astype(jnp.int32)))
            carry = tuple(nc)
        return carry

    lax.fori_loop(0, NLB, compact, tuple(offs[:NPASS]))
    for k in range(NPASS + 1):
        offsm[k] = offs[k]

    def one_pc(pc, carry):
        q = pc // NCH
        c = pc % NCH
        tab_c = tab_hbm.at[c]
        o0 = offsm[q]
        o1 = offsm[q + 1]
        lob = o0 // EB
        hib = (o1 + EB - 1) // EB

        def remap(b, _):
            for j in range(EB // 16):
                d = dstc[pl.ds(b * EB + j * 16, 16)]
                t = d - q * NH
                ok = jnp.logical_and(t >= 0, t < NH)
                idx2[0, b, pl.ds(j * 16, 16)] = jnp.where(ok, t, NH)
            return _

        lax.fori_loop(lob, hib, remap, 0)

        def act(b):
            return jnp.logical_and(b >= lob, b < hib)

        def gsl(b):
            return srcc.at[pl.ds(b * EB, EB)]

        bufs = (buf0, buf1, buf2, buf3)
        sems = (sem0, sem1, sem2, sem3)

        pltpu.sync_copy(zer_hbm.at[0], acc.at[pl.ds(sid * RPH, RPH)])
        plsc.subcore_barrier()

        for k in range(4):
            @pl.when(act(k))
            def _(k=k):
                pltpu.async_copy(tab_c.at[gsl(k)], bufs[k], sems[k])

        def body(i, carry):
            b = 4 * i
            for k in range(4):
                @pl.when(act(b + k))
                def _(k=k):
                    pltpu.make_async_copy(tab_c.at[gsl(b + k)], bufs[k],
                                          sems[k]).wait()
                    pltpu.sync_copy(bufs[k], acc.at[idx2.at[0].at[b + k]],
                                    add=True)

                @pl.when(act(b + k + 4))
                def _(k=k):
                    pltpu.async_copy(tab_c.at[gsl(b + k + 4)], bufs[k],
                                     sems[k])
            return carry

        lax.fori_loop(0, NBB // 4, body, 0)
        plsc.subcore_barrier()
        pltpu.sync_copy(
            acc.at[pl.ds(sid * RPH, RPH)],
            out_hbm.at[c].at[pl.ds(cid * NP + q * NH + sid * RPH, RPH)])
        plsc.subcore_barrier()
        return carry

    lax.fori_loop(0, NPASS * NCH, one_pc, 0)


@functools.cache
def _seg_call_fn():
    return pl.kernel(
        _seg_body,
        out_type=jax.ShapeDtypeStruct((NCH, NS, WC), jnp.float32),
        mesh=plsc.VectorSubcoreMesh(core_axis_name="c", subcore_axis_name="s"),
        compiler_params=pltpu.CompilerParams(needs_layout_passes=False),
        scratch_types=[
            pltpu.VMEM((NLB, LB), jnp.int32),
            pltpu.VMEM((PER_TILE,), jnp.int32),
            pltpu.VMEM((PER_TILE,), jnp.int32),
            pltpu.VMEM((1, NBB, EB), jnp.int32),
            pltpu.SMEM((NPASS + 1,), jnp.int32),
            pltpu.VMEM((EB, WC), jnp.float32),
            pltpu.VMEM((EB, WC), jnp.float32),
            pltpu.VMEM((EB, WC), jnp.float32),
            pltpu.VMEM((EB, WC), jnp.float32),
            pltpu.VMEM_SHARED((ACCR, WC), jnp.float32),
            pltpu.SemaphoreType.DMA,
            pltpu.SemaphoreType.DMA,
            pltpu.SemaphoreType.DMA,
            pltpu.SemaphoreType.DMA,
        ],
    )


_MB = 256


def _mm_body(x_ref, w_ref, deg_ref, out_ref, dinv_ref):
    dv = lax.rsqrt(1.0 + jnp.sum(deg_ref[0], axis=0)).reshape(_MB, 1)

    y = jnp.dot(x_ref[...], w_ref[...], preferred_element_type=jnp.float32)
    y = y * dv
    for c in range(NCH):
        out_ref[c] = y[:, c * WC:(c + 1) * WC]
    dinv_ref[...] = dv


def _mm_scale(x, w, degp):
    k = x.shape[1]
    nb = NP // _MB
    return pl.pallas_call(
        _mm_body,
        grid=(NS // _MB,),
        in_specs=[
            pl.BlockSpec((_MB, k), lambda m: (m, 0)),
            pl.BlockSpec((k, HID), lambda m: (0, 0)),
            pl.BlockSpec((1, 16, _MB), lambda m: (m // nb, 0, m % nb)),
        ],
        out_specs=[
            pl.BlockSpec((NCH, _MB, WC), lambda m: (0, m, 0)),
            pl.BlockSpec((_MB, 1), lambda m: (m, 0)),
        ],
        out_shape=[
            jax.ShapeDtypeStruct((NCH, NS, WC), jnp.float32),
            jax.ShapeDtypeStruct((NS, 1), jnp.float32),
        ],
        compiler_params=pltpu.CompilerParams(
            dimension_semantics=("arbitrary",)),
    )(x, w, degp)


def _combine_body(relu, s_ref, hw_ref, dinv_ref, b_ref, out_ref):
    m = pl.program_id(0)
    rows = m * _MB + lax.broadcasted_iota(jnp.int32, (_MB, WC), 0)
    valid = (rows % NP) < N
    dinv = dinv_ref[...]
    for c in range(NCH):
        col = s_ref[c] + hw_ref[c]
        col = dinv * col + b_ref[0, c * WC:(c + 1) * WC]
        if relu:
            col = jnp.maximum(col, 0.0)
        out_ref[:, c * WC:(c + 1) * WC] = jnp.where(valid, col, 0.0)


def _combine(s, hw, dinv, bias, relu):
    return pl.pallas_call(
        functools.partial(_combine_body, relu),
        grid=(NS // _MB,),
        in_specs=[
            pl.BlockSpec((NCH, _MB, WC), lambda m: (0, m, 0)),
            pl.BlockSpec((NCH, _MB, WC), lambda m: (0, m, 0)),
            pl.BlockSpec((_MB, 1), lambda m: (m, 0)),
            pl.BlockSpec((1, HID), lambda m: (0, 0)),
        ],
        out_specs=pl.BlockSpec((_MB, HID), lambda m: (m, 0)),
        out_shape=jax.ShapeDtypeStruct((NS, HID), jnp.float32),
        compiler_params=pltpu.CompilerParams(
            dimension_semantics=("arbitrary",)),
    )(s, hw, dinv, bias)


def _stats_body(o_ref, out_ref):
    mm = pl.program_id(1)
    blk = o_ref[...]

    @pl.when(mm == 0)
    def _():
        out_ref[...] = jnp.zeros_like(out_ref)

    out_ref[0, 0, :] += jnp.sum(blk, axis=0)
    out_ref[0, 1, :] += jnp.sum(blk * blk, axis=0)


def _stats(o):
    return pl.pallas_call(
        _stats_body,
        grid=(2, NP // _MB),
        in_specs=[pl.BlockSpec((_MB, HID),
                               lambda g, mm: (g * (NP // _MB) + mm, 0))],
        out_specs=pl.BlockSpec((1, 2, HID), lambda g, mm: (g, 0, 0)),
        out_shape=jax.ShapeDtypeStruct((2, 2, HID), jnp.float32),
        compiler_params=pltpu.CompilerParams(
            dimension_semantics=("arbitrary", "arbitrary")),
    )(o)


def _norm_body(o_ref, st_ref, out_ref):
    s = st_ref[0, 0, :]
    ss = st_ref[0, 1, :]
    mean = s / N
    var = (ss - N * mean * mean) / (N - 1)
    out_ref[...] = (o_ref[...] - mean) / jnp.sqrt(var)


def _norm(o, st):
    return pl.pallas_call(
        _norm_body,
        grid=(NS // _MB,),
        in_specs=[
            pl.BlockSpec((_MB, HID), lambda m: (m, 0)),
            pl.BlockSpec((1, 2, HID), lambda m: (m // (NP // _MB), 0, 0)),
        ],
        out_specs=pl.BlockSpec((_MB, HID), lambda m: (m, 0)),
        out_shape=jax.ShapeDtypeStruct((NS, HID), jnp.float32),
        compiler_params=pltpu.CompilerParams(
            dimension_semantics=("arbitrary",)),
    )(o, st)


def _pad_edges(a, fill):
    return jnp.concatenate([a, jnp.full((EP - E,), fill, jnp.int32)])


def kernel(x1, edge_index1, x2, edge_index2, W1, b1, W2, b2):
    x = jnp.zeros((NS, IN_DIM), jnp.float32)
    x = x.at[0:N].set(x1).at[NP:NP + N].set(x2)
    src = jnp.concatenate([_pad_edges(edge_index1[0], NP - 1),
                           _pad_edges(edge_index2[0] + NP, NS - 1)])
    dst = jnp.concatenate([_pad_edges(edge_index1[1], NP - 1),
                           _pad_edges(edge_index2[1], NP - 1)])
    pk2 = (src | (dst << 16)).reshape(1, ES // LB, LB)
    zer80 = jnp.zeros((1, DR, 128), jnp.float32)
    zerfc = jnp.zeros((1, RPH, WC), jnp.float32)
    b1r = b1.reshape(1, HID)
    b2r = b2.reshape(1, HID)

    degp = _deg_call_fn()(pk2, zer80)
    degp = degp.reshape(2, 16, NP)

    hw1, dinv = _mm_scale(x, W1, degp)
    s1 = _seg_call_fn()(pk2, hw1, zerfc)
    h = _combine(s1, hw1, dinv, b1r, True)

    hw2, _ = _mm_scale(h, W2, degp)
    s2 = _seg_call_fn()(pk2, hw2, zerfc)
    o = _combine(s2, hw2, dinv, b2r, False)

    st = _stats(o)
    z = _norm(o, st)
    return z[0:N], z[NP:NP + N]
